# baseline (speedup 1.0000x reference)
import jax
import jax.numpy as jnp
from jax import lax
from jax.experimental import pallas as pl
from jax.experimental.pallas import tpu as pltpu
from jax.experimental.pallas import tpu_sc as plsc

W = 8
NC, NS, LANES = 2, 16, 16
NW = NC * NS
N = 4096 * 200
NB = N // NW
C = 1600
G = NB // C

P5 = 0.030449004538683766
P4 = -0.1315818250887885
P3 = 0.28527268109058584
P2 = -0.4902307234234099
P1 = 0.9992354838332749
P0 = 9.975032552137188e-06


def _softplus(x):
    u = jnp.exp(-jnp.abs(x))
    p = ((((P5 * u + P4) * u + P3) * u + P2) * u + P1) * u + P0
    return jnp.maximum(x, 0.0) + p


def _body(ids_hbm, tab_hbm, out_hbm, idx_v, obuf, sem):
    wid = lax.axis_index("s") * NC + lax.axis_index("c")
    base = wid * NB
    iota = lax.iota(jnp.int32, LANES)
    fmask = iota < 8

    def chunk(g, _):
        off = base + g * C
        pltpu.sync_copy(ids_hbm.at[pl.ds(off, C)], idx_v)
        pltpu.async_copy(tab_hbm.at[idx_v], obuf, sem).wait()

        def row(i, _):
            x = obuf[i, :]
            obuf[i, :] = jnp.where(fmask, _softplus(x), x)
            return 0

        lax.fori_loop(0, C, row, 0)
        pltpu.sync_copy(obuf, out_hbm.at[pl.ds(off, C)])
        return 0

    lax.fori_loop(0, G, chunk, 0)


@jax.jit
def kernel(token_ids, frequencies, amplitudes):
    B, L = token_ids.shape
    ids = token_ids.reshape(N)
    tab = jnp.concatenate([frequencies, amplitudes], axis=1)
    mesh = plsc.VectorSubcoreMesh(core_axis_name="c", subcore_axis_name="s",
                                  num_cores=NC, num_subcores=NS)
    out = pl.kernel(
        _body,
        out_type=jax.ShapeDtypeStruct((N, 2 * W), jnp.float32),
        mesh=mesh,
        scratch_types=[
            pltpu.VMEM((C,), jnp.int32),
            pltpu.VMEM((C, 2 * W), jnp.float32),
            pltpu.SemaphoreType.DMA,
        ],
        compiler_params=pltpu.CompilerParams(use_tc_tiling_on_sc=False),
    )(ids, tab)
    return out.reshape(B, L, 2 * W)


# speedup vs baseline: 1.3010x; 1.3010x over previous
import jax
import jax.numpy as jnp
from jax import lax
from jax.experimental import pallas as pl
from jax.experimental.pallas import tpu as pltpu
from jax.experimental.pallas import tpu_sc as plsc

W = 8
D = 2 * W
L = 200
NC, NS, LANES = 2, 16, 16
NW = NC * NS
N = 4096 * L
NB = N // NW
SB = NB // L
CSEQ = 4
CS = CSEQ * L
G = NB // CS
HALF = G // 2

P5 = 0.030449004538683766
P4 = -0.1315818250887885
P3 = 0.28527268109058584
P2 = -0.4902307234234099
P1 = 0.9992354838332749
P0 = 9.975032552137188e-06


def _softplus(x):
    u = jnp.exp(-jnp.abs(x))
    p = ((((P5 * u + P4) * u + P3) * u + P2) * u + P1) * u + P0
    return jnp.maximum(x, 0.0) + p


def _body(ids_hbm, tab_hbm, out_hbm, idxall, ob0, ob1,
          sg0, sg1, so0, so1):
    wid = lax.axis_index("s") * NC + lax.axis_index("c")
    base = wid * NB
    s_base = wid * SB
    iota = lax.iota(jnp.int32, LANES)
    fmask = iota < W

    pltpu.sync_copy(ids_hbm.at[pl.ds(base, NB)], idxall)

    def gather(g, ob, sg):
        for j in range(CSEQ):
            pltpu.async_copy(
                tab_hbm.at[idxall.at[pl.ds(g * CS + j * L, L)]],
                ob.at[j], sg)

    def drain_gather(g, ob, sg):
        for j in range(CSEQ):
            pltpu.make_async_copy(
                tab_hbm.at[idxall.at[pl.ds(g * CS + j * L, L)]],
                ob.at[j], sg).wait()

    def owin(g):
        return out_hbm.at[pl.ds(s_base + g * CSEQ, CSEQ)]

    def compute(ob):
        for j in range(CSEQ):
            def rows(i, _, j=j):
                x0 = ob[j, 2 * i, :]
                x1 = ob[j, 2 * i + 1, :]
                ob[j, 2 * i, :] = jnp.where(fmask, _softplus(x0), x0)
                ob[j, 2 * i + 1, :] = jnp.where(fmask, _softplus(x1), x1)
                return 0
            lax.fori_loop(0, L // 2, rows, 0)

    gather(0, ob0, sg0)

    def step(p, _):
        g0 = 2 * p

        @pl.when(p > 0)
        def _():
            pltpu.make_async_copy(ob1, owin(g0 - 1), so1).wait()

        gather(g0 + 1, ob1, sg1)
        drain_gather(g0, ob0, sg0)
        compute(ob0)
        pltpu.async_copy(ob0, owin(g0), so0)

        @pl.when(p < HALF - 1)
        def _():
            pltpu.make_async_copy(ob0, owin(g0), so0).wait()
            gather(g0 + 2, ob0, sg0)

        drain_gather(g0 + 1, ob1, sg1)
        compute(ob1)
        pltpu.async_copy(ob1, owin(g0 + 1), so1)
        return 0

    lax.fori_loop(0, HALF, step, 0)
    pltpu.make_async_copy(ob0, owin(G - 2), so0).wait()
    pltpu.make_async_copy(ob1, owin(G - 1), so1).wait()


@jax.jit
def kernel(token_ids, frequencies, amplitudes):
    B, LL = token_ids.shape
    ids = token_ids.reshape(N)
    tab = jnp.concatenate([frequencies, amplitudes], axis=1)
    mesh = plsc.VectorSubcoreMesh(core_axis_name="c", subcore_axis_name="s",
                                  num_cores=NC, num_subcores=NS)
    out = pl.kernel(
        _body,
        out_type=jax.ShapeDtypeStruct((B, LL, D), jnp.float32),
        mesh=mesh,
        scratch_types=[
            pltpu.VMEM((NB,), jnp.int32),
            pltpu.VMEM((CSEQ, L, D), jnp.float32),
            pltpu.VMEM((CSEQ, L, D), jnp.float32),
            pltpu.SemaphoreType.DMA,
            pltpu.SemaphoreType.DMA,
            pltpu.SemaphoreType.DMA,
            pltpu.SemaphoreType.DMA,
        ],
        compiler_params=pltpu.CompilerParams(use_tc_tiling_on_sc=False),
    )(ids, tab)
    return out
